# Initial kernel scaffold; baseline (speedup 1.0000x reference)
import jax
import jax.numpy as jnp
from jax.experimental import pallas as pl


def kernel(response, elapsed_time, response_table, pos_table, elapsed_W):
    raise NotImplementedError("write your pallas kernel here")


# speedup vs baseline: 2.1663x; 2.1663x over previous
import functools

import jax
import jax.numpy as jnp
from jax import lax
from jax.experimental import pallas as pl
from jax.experimental.pallas import tpu as pltpu
from jax.experimental.pallas import tpu_sc as plsc

D = 64
L_SEQ = 200
N_CORES = 2
N_SUBCORES = 16
N_WORKERS = N_CORES * N_SUBCORES
CHUNK = 512
SUB = 128
LANES = 16


def _sc_call(n_rows):
    rows_per_w = n_rows // N_WORKERS
    n_chunks = rows_per_w // CHUNK
    assert rows_per_w * N_WORKERS == n_rows
    assert n_chunks * CHUNK == rows_per_w

    def body(idx_hbm, e_hbm, table_hbm, pos_hbm, w_hbm, out_hbm,
             idx_v, e_v, rows_v, pos_v, w_v, sem):
        wid = lax.axis_index("s") * N_CORES + lax.axis_index("c")
        pltpu.sync_copy(pos_hbm, pos_v)
        pltpu.sync_copy(w_hbm, w_v)
        w_vecs = [w_v[pl.ds(LANES * j, LANES)] for j in range(D // LANES)]
        base_w = wid * rows_per_w

        def chunk_body(c, carry):
            base = base_w + c * CHUNK
            pltpu.sync_copy(idx_hbm.at[pl.ds(base, CHUNK)], idx_v)
            pltpu.sync_copy(e_hbm.at[pl.ds(base, CHUNK)], e_v)
            handles = [
                pltpu.async_copy(
                    table_hbm.at[idx_v.at[pl.ds(s * SUB, SUB)]],
                    rows_v.at[pl.ds(s * SUB, SUB)],
                    sem,
                )
                for s in range(CHUNK // SUB)
            ]
            for h in handles:
                h.wait()

            def group_body(g, gcarry):
                r0 = g * LANES
                e16 = e_v[pl.ds(r0, LANES)]
                for i in range(LANES):
                    r = r0 + i
                    l = lax.rem(base + r, L_SEQ)
                    ev = jnp.full((LANES,), e16[i], jnp.float32)
                    for j in range(D // LANES):
                        sl = pl.ds(LANES * j, LANES)
                        rows_v[r, sl] = rows_v[r, sl] + pos_v[l, sl] + ev * w_vecs[j]
                return gcarry

            lax.fori_loop(0, CHUNK // LANES, group_body, 0)
            pltpu.sync_copy(rows_v, out_hbm.at[pl.ds(base, CHUNK)])
            return carry

        lax.fori_loop(0, n_chunks, chunk_body, 0)

    return pl.kernel(
        body,
        out_type=jax.ShapeDtypeStruct((n_rows, D), jnp.float32),
        mesh=plsc.VectorSubcoreMesh(core_axis_name="c", subcore_axis_name="s"),
        compiler_params=pltpu.CompilerParams(use_tc_tiling_on_sc=False),
        scratch_types=[
            pltpu.VMEM((CHUNK,), jnp.int32),
            pltpu.VMEM((CHUNK,), jnp.float32),
            pltpu.VMEM((CHUNK, D), jnp.float32),
            pltpu.VMEM((L_SEQ, D), jnp.float32),
            pltpu.VMEM((D,), jnp.float32),
            pltpu.SemaphoreType.DMA,
        ],
    )


def kernel(response, elapsed_time, response_table, pos_table, elapsed_W):
    batch, seq_len = response.shape
    n_rows = batch * seq_len
    idx = response.reshape(n_rows)
    e_flat = elapsed_time.reshape(n_rows)
    w_flat = elapsed_W.reshape(D)
    out = _sc_call(n_rows)(idx, e_flat, response_table, pos_table, w_flat)
    return out.reshape(batch, seq_len, D)


# speedup vs baseline: 2.5650x; 1.1841x over previous
import jax
import jax.numpy as jnp
from jax import lax
from jax.experimental import pallas as pl
from jax.experimental.pallas import tpu as pltpu
from jax.experimental.pallas import tpu_sc as plsc

D = 64
L_SEQ = 200
N_CORES = 2
N_SUBCORES = 16
N_WORKERS = N_CORES * N_SUBCORES
CHUNK = 512
SUB = 128
LANES = 16
NBUF = 2


def _sc_call(n_rows):
    rows_per_w = n_rows // N_WORKERS
    n_chunks = rows_per_w // CHUNK
    assert rows_per_w * N_WORKERS == n_rows
    assert n_chunks * CHUNK == rows_per_w
    assert n_chunks % NBUF == 0

    def body(idx_hbm, e_hbm, table_hbm, pos_hbm, w_hbm, out_hbm,
             idx_v, e_v, rows_v, pos2_v, w_v, sem_g0, sem_g1, sem_w):
        wid = lax.axis_index("s") * N_CORES + lax.axis_index("c")
        pltpu.sync_copy(pos_hbm, pos2_v.at[pl.ds(0, L_SEQ)])
        pltpu.sync_copy(pos_hbm, pos2_v.at[pl.ds(L_SEQ, L_SEQ)])
        pltpu.sync_copy(w_hbm, w_v)
        w_vecs = [w_v[pl.ds(LANES * j, LANES)] for j in range(D // LANES)]
        sems = (sem_g0, sem_g1)
        base_w = wid * rows_per_w

        def fire(c, b):
            base = base_w + c * CHUNK
            pltpu.sync_copy(idx_hbm.at[pl.ds(base, CHUNK)], idx_v.at[b])
            pltpu.sync_copy(e_hbm.at[pl.ds(base, CHUNK)], e_v.at[b])
            for s in range(CHUNK // SUB):
                pltpu.async_copy(
                    table_hbm.at[idx_v.at[b, pl.ds(s * SUB, SUB)]],
                    rows_v.at[b, pl.ds(s * SUB, SUB)],
                    sems[b],
                )

        fire(0, 0)

        def outer(c2, ph0):
            for b in range(NBUF):
                c = c2 * NBUF + b
                nb = 1 - b

                @pl.when(c >= 1)
                def _():
                    pltpu.make_async_copy(
                        rows_v.at[nb], out_hbm.at[pl.ds(0, CHUNK)], sem_w
                    ).wait()

                @pl.when(c + 1 < n_chunks)
                def _():
                    fire(c + 1, nb)

                pltpu.make_async_copy(
                    out_hbm.at[pl.ds(0, CHUNK)], rows_v.at[b], sems[b]
                ).wait()

                def group(g, ph):
                    e16 = e_v[b, pl.ds(g * LANES, LANES)]
                    for i in range(LANES):
                        r = g * LANES + i
                        ev = jnp.full((LANES,), e16[i], jnp.float32)
                        for j in range(D // LANES):
                            sl = pl.ds(LANES * j, LANES)
                            t = pos2_v[ph + i, sl] + ev * w_vecs[j]
                            plsc.addupdate(rows_v.at[b, r, sl], t)
                    phn = ph + LANES
                    return jnp.where(phn >= L_SEQ, phn - L_SEQ, phn)

                lax.fori_loop(0, CHUNK // LANES, group, ph0)

                base = base_w + c * CHUNK
                pltpu.async_copy(
                    rows_v.at[b], out_hbm.at[pl.ds(base, CHUNK)], sem_w
                )
                ph0 = ph0 + (CHUNK % L_SEQ)
                ph0 = jnp.where(ph0 >= L_SEQ, ph0 - L_SEQ, ph0)
            return ph0

        lax.fori_loop(0, n_chunks // NBUF, outer, jnp.int32(0))
        pltpu.make_async_copy(
            rows_v.at[(n_chunks - 1) % NBUF], out_hbm.at[pl.ds(0, CHUNK)], sem_w
        ).wait()

    return pl.kernel(
        body,
        out_type=jax.ShapeDtypeStruct((n_rows, D), jnp.float32),
        mesh=plsc.VectorSubcoreMesh(core_axis_name="c", subcore_axis_name="s"),
        compiler_params=pltpu.CompilerParams(use_tc_tiling_on_sc=False),
        scratch_types=[
            pltpu.VMEM((NBUF, CHUNK), jnp.int32),
            pltpu.VMEM((NBUF, CHUNK), jnp.float32),
            pltpu.VMEM((NBUF, CHUNK, D), jnp.float32),
            pltpu.VMEM((2 * L_SEQ, D), jnp.float32),
            pltpu.VMEM((D,), jnp.float32),
            pltpu.SemaphoreType.DMA,
            pltpu.SemaphoreType.DMA,
            pltpu.SemaphoreType.DMA,
        ],
    )


def kernel(response, elapsed_time, response_table, pos_table, elapsed_W):
    batch, seq_len = response.shape
    n_rows = batch * seq_len
    idx = response.reshape(n_rows)
    e_flat = elapsed_time.reshape(n_rows)
    w_flat = elapsed_W.reshape(D)
    out = _sc_call(n_rows)(idx, e_flat, response_table, pos_table, w_flat)
    return out.reshape(batch, seq_len, D)
